# Initial kernel scaffold; baseline (speedup 1.0000x reference)
import jax
import jax.numpy as jnp
from jax.experimental import pallas as pl


def kernel(x, gate_w1, gate_b1, gate_w2, gate_b2, expert_w, expert_b):
    raise NotImplementedError("write your pallas kernel here")


# speedup vs baseline: 1.7966x; 1.7966x over previous
import functools

import jax
import jax.numpy as jnp
from jax.experimental import pallas as pl
from jax.experimental.pallas import tpu as pltpu

B, C, L = 32, 256, 2048
D4 = 192
E = 16
K = 2


def _fused_kernel(x_ref, w1_ref, b1_ref, w2_ref, b2_ref, out_ref):
    b = pl.program_id(0)
    xb = x_ref[0]
    h = jnp.maximum(
        jnp.dot(xb, w1_ref[...], preferred_element_type=jnp.float32) + b1_ref[...],
        0.0,
    )
    logits = jnp.dot(h, w2_ref[...], preferred_element_type=jnp.float32) + b2_ref[...]

    m = jnp.max(logits, axis=1, keepdims=True)
    ex = jnp.exp(logits - m)
    p = ex / jnp.sum(ex, axis=1, keepdims=True)

    lane = jax.lax.broadcasted_iota(jnp.int32, (C, E), 1)
    p1 = jnp.max(p, axis=1, keepdims=True)
    i1 = jnp.min(jnp.where(p == p1, lane, E), axis=1, keepdims=True)
    pm = jnp.where(lane == i1, -jnp.inf, p)
    p2 = jnp.max(pm, axis=1, keepdims=True)
    i2 = jnp.min(jnp.where(pm == p2, lane, E), axis=1, keepdims=True)

    s = p1 + p2 + 1e-6
    g = jnp.where(lane == i1, p1 / s, 0.0) + jnp.where(lane == i2, p2 / s, 0.0)

    mask_b = jax.lax.dot_general(
        g, g, (((1,), (1,)), ((), ())), preferred_element_type=jnp.float32
    )

    @pl.when(b == 0)
    def _init():
        out_ref[...] = mask_b * (1.0 / B)

    @pl.when(b != 0)
    def _acc():
        out_ref[...] += mask_b * (1.0 / B)


@jax.jit
def kernel(x, gate_w1, gate_b1, gate_w2, gate_b2, expert_w, expert_b):
    del expert_w, expert_b
    b1 = gate_b1.reshape(1, D4)
    b2 = gate_b2.reshape(1, E)
    return pl.pallas_call(
        _fused_kernel,
        grid=(B,),
        in_specs=[
            pl.BlockSpec((1, C, L), lambda b: (b, 0, 0)),
            pl.BlockSpec((L, D4), lambda b: (0, 0)),
            pl.BlockSpec((1, D4), lambda b: (0, 0)),
            pl.BlockSpec((D4, E), lambda b: (0, 0)),
            pl.BlockSpec((1, E), lambda b: (0, 0)),
        ],
        out_specs=pl.BlockSpec((C, C), lambda b: (0, 0)),
        out_shape=jax.ShapeDtypeStruct((C, C), jnp.float32),
        compiler_params=pltpu.CompilerParams(
            dimension_semantics=("arbitrary",),
        ),
    )(x, gate_w1, b1, gate_w2, b2)


# speedup vs baseline: 2.9595x; 1.6472x over previous
import functools

import jax
import jax.numpy as jnp
from jax.experimental import pallas as pl
from jax.experimental.pallas import tpu as pltpu

B, C, L = 32, 256, 2048
D4 = 192
E = 16
K = 2


BPB = 4


def _fused_kernel(x_ref, w1_ref, b1_ref, w2_ref, b2_ref, out_ref):
    step = pl.program_id(0)
    xb = x_ref[...].reshape(BPB * C, L)
    h = jnp.maximum(
        jnp.dot(xb, w1_ref[...], preferred_element_type=jnp.float32) + b1_ref[...],
        0.0,
    )
    logits = jnp.dot(h, w2_ref[...], preferred_element_type=jnp.float32) + b2_ref[...]

    m = jnp.max(logits, axis=1, keepdims=True)
    ex = jnp.exp(logits - m)
    p = ex / jnp.sum(ex, axis=1, keepdims=True)

    lane = jax.lax.broadcasted_iota(jnp.int32, (BPB * C, E), 1)
    p1 = jnp.max(p, axis=1, keepdims=True)
    i1 = jnp.min(jnp.where(p == p1, lane, E), axis=1, keepdims=True)
    pm = jnp.where(lane == i1, -jnp.inf, p)
    p2 = jnp.max(pm, axis=1, keepdims=True)
    i2 = jnp.min(jnp.where(pm == p2, lane, E), axis=1, keepdims=True)

    s = p1 + p2 + 1e-6
    g = jnp.where(lane == i1, p1 / s, 0.0) + jnp.where(lane == i2, p2 / s, 0.0)

    acc = None
    for j in range(BPB):
        gj = g[j * C:(j + 1) * C, :]
        mask_b = jax.lax.dot_general(
            gj, gj, (((1,), (1,)), ((), ())), preferred_element_type=jnp.float32
        )
        acc = mask_b if acc is None else acc + mask_b

    @pl.when(step == 0)
    def _init():
        out_ref[...] = acc * (1.0 / B)

    @pl.when(step != 0)
    def _acc():
        out_ref[...] += acc * (1.0 / B)


@jax.jit
def kernel(x, gate_w1, gate_b1, gate_w2, gate_b2, expert_w, expert_b):
    del expert_w, expert_b
    b1 = gate_b1.reshape(1, D4)
    b2 = gate_b2.reshape(1, E)
    return pl.pallas_call(
        _fused_kernel,
        grid=(B // BPB,),
        in_specs=[
            pl.BlockSpec((BPB, C, L), lambda b: (b, 0, 0)),
            pl.BlockSpec((L, D4), lambda b: (0, 0)),
            pl.BlockSpec((1, D4), lambda b: (0, 0)),
            pl.BlockSpec((D4, E), lambda b: (0, 0)),
            pl.BlockSpec((1, E), lambda b: (0, 0)),
        ],
        out_specs=pl.BlockSpec((C, C), lambda b: (0, 0)),
        out_shape=jax.ShapeDtypeStruct((C, C), jnp.float32),
        compiler_params=pltpu.CompilerParams(
            dimension_semantics=("arbitrary",),
        ),
    )(x, gate_w1, b1, gate_w2, b2)


# speedup vs baseline: 3.1133x; 1.0520x over previous
import functools

import jax
import jax.numpy as jnp
from jax.experimental import pallas as pl
from jax.experimental.pallas import tpu as pltpu

B, C, L = 32, 256, 2048
D4 = 192
E = 16
K = 2


BPB = 8


def _fused_kernel(x_ref, w1_ref, b1_ref, w2_ref, b2_ref, out_ref):
    step = pl.program_id(0)
    xb = x_ref[...].reshape(BPB * C, L)
    h = jnp.maximum(
        jnp.dot(xb, w1_ref[...], preferred_element_type=jnp.float32) + b1_ref[...],
        0.0,
    )
    logits = jnp.dot(h, w2_ref[...], preferred_element_type=jnp.float32) + b2_ref[...]

    m = jnp.max(logits, axis=1, keepdims=True)
    ex = jnp.exp(logits - m)
    p = ex / jnp.sum(ex, axis=1, keepdims=True)

    lane = jax.lax.broadcasted_iota(jnp.int32, (BPB * C, E), 1)
    p1 = jnp.max(p, axis=1, keepdims=True)
    i1 = jnp.min(jnp.where(p == p1, lane, E), axis=1, keepdims=True)
    pm = jnp.where(lane == i1, -jnp.inf, p)
    p2 = jnp.max(pm, axis=1, keepdims=True)
    i2 = jnp.min(jnp.where(pm == p2, lane, E), axis=1, keepdims=True)

    s = p1 + p2 + 1e-6
    g = jnp.where(lane == i1, p1 / s, 0.0) + jnp.where(lane == i2, p2 / s, 0.0)

    acc = None
    for j in range(BPB):
        gj = g[j * C:(j + 1) * C, :]
        mask_b = jax.lax.dot_general(
            gj, gj, (((1,), (1,)), ((), ())), preferred_element_type=jnp.float32
        )
        acc = mask_b if acc is None else acc + mask_b

    @pl.when(step == 0)
    def _init():
        out_ref[...] = acc * (1.0 / B)

    @pl.when(step != 0)
    def _acc():
        out_ref[...] += acc * (1.0 / B)


@jax.jit
def kernel(x, gate_w1, gate_b1, gate_w2, gate_b2, expert_w, expert_b):
    del expert_w, expert_b
    b1 = gate_b1.reshape(1, D4)
    b2 = gate_b2.reshape(1, E)
    return pl.pallas_call(
        _fused_kernel,
        grid=(B // BPB,),
        in_specs=[
            pl.BlockSpec((BPB, C, L), lambda b: (b, 0, 0)),
            pl.BlockSpec((L, D4), lambda b: (0, 0)),
            pl.BlockSpec((1, D4), lambda b: (0, 0)),
            pl.BlockSpec((D4, E), lambda b: (0, 0)),
            pl.BlockSpec((1, E), lambda b: (0, 0)),
        ],
        out_specs=pl.BlockSpec((C, C), lambda b: (0, 0)),
        out_shape=jax.ShapeDtypeStruct((C, C), jnp.float32),
        compiler_params=pltpu.CompilerParams(
            dimension_semantics=("arbitrary",),
        ),
    )(x, gate_w1, b1, gate_w2, b2)
